# baseline (speedup 1.0000x reference)
import functools

import jax
import jax.numpy as jnp
import numpy as np
from jax import lax
from jax.experimental import pallas as pl
from jax.experimental.pallas import tpu as pltpu
from jax.experimental.pallas import tpu_sc as plsc

EMBED_DIM = 256
IMG_EMB_SIZE = 32


def _dense_body(x_ref, w1_ref, b1_ref, w2_ref, b2_ref,
                w3_ref, b3_ref, out_ref):
    xf = x_ref[0]
    p2k = jnp.maximum(
        jnp.dot(w1_ref[...], xf, preferred_element_type=jnp.float32)
        + b1_ref[...], 0.0)
    h2 = jnp.maximum(
        jnp.dot(w2_ref[...], p2k, preferred_element_type=jnp.float32)
        + b2_ref[...], 0.0)
    out_ref[0] = (jnp.dot(w3_ref[...], h2, preferred_element_type=jnp.float32)
                  + b3_ref[...])


def _dense_call(masks, conv1_w, conv1_b, conv2_w, conv2_b, conv3_w, conv3_b,
                interpret=False):
    B = masks.shape[0]
    xf = masks.reshape(B, 32, 4, 32, 4).transpose(0, 2, 4, 1, 3)
    xf = xf.reshape(B, 16, 1024)

    w1c = conv1_w[:, 0]
    w1big = jnp.stack([
        jnp.pad(w1c, ((0, 0), (2 * di, 2 - 2 * di), (2 * dj, 2 - 2 * dj)))
        for di in range(2) for dj in range(2)
    ], axis=0).reshape(256, 16)
    b1col = jnp.tile(conv1_b, (4,)).reshape(256, 1)
    w2m = conv2_w.transpose(0, 2, 3, 1).reshape(256, 256)
    b2col = conv2_b.reshape(256, 1)
    w3m = conv3_w[:, :, 0, 0]
    b3col = conv3_b.reshape(256, 1)

    const = lambda *_: (0, 0)
    out = pl.pallas_call(
        _dense_body,
        grid=(B,),
        in_specs=[
            pl.BlockSpec((1, 16, 1024), lambda i: (i, 0, 0)),
            pl.BlockSpec((256, 16), const),
            pl.BlockSpec((256, 1), const),
            pl.BlockSpec((256, 256), const),
            pl.BlockSpec((256, 1), const),
            pl.BlockSpec((256, 256), const),
            pl.BlockSpec((256, 1), const),
        ],
        out_specs=pl.BlockSpec((1, 256, 1024), lambda i: (i, 0, 0)),
        out_shape=jax.ShapeDtypeStruct((B, 256, 1024), jnp.float32),
        interpret=interpret,
    )(xf, w1big, b1col, w2m, b2col, w3m, b3col)
    return out.reshape(B, 256, 32, 32)


_NPOINT = 32
_NBOX = 8
_NSLOT = _NPOINT + _NBOX


def _fuse_body(pe_ref, rows_ref, out_ref):
    mul = jnp.where(pl.program_id(0) == 2, 0.0, 1.0)
    out_ref[0] = pe_ref[...] * mul + rows_ref[0]


def _fuse_table(pe_flat, point_emb_w, box_emb_w, interpret=False):
    rows = jnp.concatenate([point_emb_w, box_emb_w], axis=0)
    rows = rows.reshape(3, 1, EMBED_DIM)
    fused = pl.pallas_call(
        _fuse_body,
        grid=(3,),
        in_specs=[
            pl.BlockSpec((1024, EMBED_DIM), lambda r: (0, 0)),
            pl.BlockSpec((1, 1, EMBED_DIM), lambda r: (r, 0, 0)),
        ],
        out_specs=pl.BlockSpec((1, 1024, EMBED_DIM), lambda r: (r, 0, 0)),
        out_shape=jax.ShapeDtypeStruct((3, 1024, EMBED_DIM), jnp.float32),
        interpret=interpret,
    )(pe_flat, rows)
    return fused.reshape(3 * 1024, EMBED_DIM)


def _sparse_body(pk_hbm, table_hbm, out_hbm, pk_v, idx_v, buf_v, sem):
    nc = 2
    wid = lax.axis_index("s") * nc + lax.axis_index("c")
    pltpu.sync_copy(pk_hbm.at[pl.ds(wid * 192, 192)], pk_v)
    scale = jnp.float32(IMG_EMB_SIZE / 512.0)
    smax = IMG_EMB_SIZE - 1
    box_idx = jnp.full((16,), 2048, jnp.int32)

    def point_chunk(k):
        sl = pl.ds(k * 16, 16)
        xv = pk_v[sl]
        yv = pk_v[pl.ds(64 + k * 16, 16)]
        lv = pk_v[pl.ds(128 + k * 16, 16)].astype(jnp.int32)
        xi = jnp.clip((xv * scale).astype(jnp.int32), 0, smax)
        yi = jnp.clip((yv * scale).astype(jnp.int32), 0, smax)
        return lv * 1024 + yi * IMG_EMB_SIZE + xi

    idx_v[pl.ds(0, 16)] = point_chunk(0)
    idx_v[pl.ds(16, 16)] = point_chunk(1)
    idx_v[pl.ds(32, 16)] = box_idx
    idx_v[pl.ds(40, 16)] = point_chunk(2)
    idx_v[pl.ds(64, 16)] = box_idx
    idx_v[pl.ds(56, 16)] = point_chunk(3)
    copies = [
        pltpu.async_copy(table_hbm.at[idx_v.at[pl.ds(k * 16, 16)]],
                         buf_v.at[pl.ds(k * 16, 16)], sem)
        for k in range(5)
    ]
    for c in copies:
        c.wait()
    pltpu.sync_copy(buf_v, out_hbm.at[pl.ds(wid * 80, 80)])


def _sparse_call(point_coords, point_labels, point_emb_w, box_emb_w, pe_layer):
    B, Np = point_labels.shape
    nw = 32
    rows_per_w = B * _NSLOT // nw
    npt = B * Np // nw
    xs = point_coords[..., 0].reshape(nw, 1, npt)
    ys = point_coords[..., 1].reshape(nw, 1, npt)
    lab = point_labels.astype(jnp.float32).reshape(nw, 1, npt)
    packed = jnp.concatenate([xs, ys, lab], axis=1).reshape(nw * 3 * npt)
    table = _fuse_table(pe_layer.reshape(1024, EMBED_DIM), point_emb_w,
                        box_emb_w)

    mesh = plsc.VectorSubcoreMesh(core_axis_name="c", subcore_axis_name="s")
    out = pl.kernel(
        _sparse_body,
        out_type=jax.ShapeDtypeStruct((B * _NSLOT, EMBED_DIM), jnp.float32),
        mesh=mesh,
        scratch_types=[
            pltpu.VMEM((3 * npt,), jnp.float32),
            pltpu.VMEM((rows_per_w,), jnp.int32),
            pltpu.VMEM((rows_per_w, EMBED_DIM), jnp.float32),
            pltpu.SemaphoreType.DMA,
        ],
    )(packed, table)
    return out.reshape(B, _NSLOT, EMBED_DIM)


def kernel(point_coords, point_labels, boxes, masks, point_emb_w, box_emb_w,
           conv1_w, conv1_b, conv2_w, conv2_b, conv3_w, conv3_b, pe_layer):
    sparse_embed = _sparse_call(point_coords, point_labels, point_emb_w,
                                box_emb_w, pe_layer)
    dense_embed = _dense_call(masks, conv1_w, conv1_b, conv2_w, conv2_b,
                              conv3_w, conv3_b)
    return (sparse_embed, dense_embed)


# speedup vs baseline: 1.0844x; 1.0844x over previous
import functools

import jax
import jax.numpy as jnp
import numpy as np
from jax import lax
from jax.experimental import pallas as pl
from jax.experimental.pallas import tpu as pltpu
from jax.experimental.pallas import tpu_sc as plsc

EMBED_DIM = 256
IMG_EMB_SIZE = 32


def _dense_body(x_ref, w1_ref, b1_ref, w2_ref, b2_ref,
                w3_ref, b3_ref, out_ref):
    for b in range(x_ref.shape[0]):
        xf = x_ref[b]
        p2k = jnp.maximum(
            jnp.dot(w1_ref[...], xf, preferred_element_type=jnp.float32)
            + b1_ref[...], 0.0)
        h2 = jnp.maximum(
            jnp.dot(w2_ref[...], p2k, preferred_element_type=jnp.float32)
            + b2_ref[...], 0.0)
        out_ref[b] = (jnp.dot(w3_ref[...], h2,
                              preferred_element_type=jnp.float32)
                      + b3_ref[...])


def _dense_call(masks, conv1_w, conv1_b, conv2_w, conv2_b, conv3_w, conv3_b,
                interpret=False):
    B = masks.shape[0]
    xf = masks.reshape(B, 32, 4, 32, 4).transpose(0, 2, 4, 1, 3)
    xf = xf.reshape(B, 16, 1024)

    w1c = conv1_w[:, 0]
    w1big = jnp.stack([
        jnp.pad(w1c, ((0, 0), (2 * di, 2 - 2 * di), (2 * dj, 2 - 2 * dj)))
        for di in range(2) for dj in range(2)
    ], axis=0).reshape(256, 16)
    b1col = jnp.tile(conv1_b, (4,)).reshape(256, 1)
    w2m = conv2_w.transpose(0, 2, 3, 1).reshape(256, 256)
    b2col = conv2_b.reshape(256, 1)
    w3m = conv3_w[:, :, 0, 0]
    b3col = conv3_b.reshape(256, 1)

    const = lambda *_: (0, 0)
    bpb = 2
    out = pl.pallas_call(
        _dense_body,
        grid=(B // bpb,),
        in_specs=[
            pl.BlockSpec((bpb, 16, 1024), lambda i: (i, 0, 0)),
            pl.BlockSpec((256, 16), const),
            pl.BlockSpec((256, 1), const),
            pl.BlockSpec((256, 256), const),
            pl.BlockSpec((256, 1), const),
            pl.BlockSpec((256, 256), const),
            pl.BlockSpec((256, 1), const),
        ],
        out_specs=pl.BlockSpec((bpb, 256, 1024), lambda i: (i, 0, 0)),
        out_shape=jax.ShapeDtypeStruct((B, 256, 1024), jnp.float32),
        interpret=interpret,
    )(xf, w1big, b1col, w2m, b2col, w3m, b3col)
    return out.reshape(B, 256, 32, 32)


_NPOINT = 32
_NBOX = 8
_NSLOT = _NPOINT + _NBOX


def _fuse_body(pe_ref, rows_ref, out_ref):
    mul = jnp.where(pl.program_id(0) == 2, 0.0, 1.0)
    out_ref[0] = pe_ref[...] * mul + rows_ref[0]


def _fuse_table(pe_flat, point_emb_w, box_emb_w, interpret=False):
    rows = jnp.concatenate([point_emb_w, box_emb_w], axis=0)
    rows = rows.reshape(3, 1, EMBED_DIM)
    fused = pl.pallas_call(
        _fuse_body,
        grid=(3,),
        in_specs=[
            pl.BlockSpec((1024, EMBED_DIM), lambda r: (0, 0)),
            pl.BlockSpec((1, 1, EMBED_DIM), lambda r: (r, 0, 0)),
        ],
        out_specs=pl.BlockSpec((1, 1024, EMBED_DIM), lambda r: (r, 0, 0)),
        out_shape=jax.ShapeDtypeStruct((3, 1024, EMBED_DIM), jnp.float32),
        interpret=interpret,
    )(pe_flat, rows)
    return fused.reshape(3 * 1024, EMBED_DIM)


def _sparse_body(pk_hbm, table_hbm, out_hbm, pk_v, idx_v, buf_v, sem):
    nc = 2
    wid = lax.axis_index("s") * nc + lax.axis_index("c")
    pltpu.sync_copy(pk_hbm.at[pl.ds(wid * 192, 192)], pk_v)
    scale = jnp.float32(IMG_EMB_SIZE / 512.0)
    smax = IMG_EMB_SIZE - 1
    box_idx = jnp.full((16,), 2048, jnp.int32)

    def point_chunk(k):
        sl = pl.ds(k * 16, 16)
        xv = pk_v[sl]
        yv = pk_v[pl.ds(64 + k * 16, 16)]
        lv = pk_v[pl.ds(128 + k * 16, 16)].astype(jnp.int32)
        xi = jnp.clip((xv * scale).astype(jnp.int32), 0, smax)
        yi = jnp.clip((yv * scale).astype(jnp.int32), 0, smax)
        return lv * 1024 + yi * IMG_EMB_SIZE + xi

    idx_v[pl.ds(0, 16)] = point_chunk(0)
    idx_v[pl.ds(16, 16)] = point_chunk(1)
    idx_v[pl.ds(32, 16)] = box_idx
    idx_v[pl.ds(40, 16)] = point_chunk(2)
    idx_v[pl.ds(64, 16)] = box_idx
    idx_v[pl.ds(56, 16)] = point_chunk(3)
    copies = [
        pltpu.async_copy(table_hbm.at[idx_v.at[pl.ds(k * 16, 16)]],
                         buf_v.at[pl.ds(k * 16, 16)], sem)
        for k in range(5)
    ]
    for c in copies:
        c.wait()
    pltpu.sync_copy(buf_v, out_hbm.at[pl.ds(wid * 80, 80)])


def _sparse_call(point_coords, point_labels, point_emb_w, box_emb_w, pe_layer):
    B, Np = point_labels.shape
    nw = 32
    rows_per_w = B * _NSLOT // nw
    npt = B * Np // nw
    xs = point_coords[..., 0].reshape(nw, 1, npt)
    ys = point_coords[..., 1].reshape(nw, 1, npt)
    lab = point_labels.astype(jnp.float32).reshape(nw, 1, npt)
    packed = jnp.concatenate([xs, ys, lab], axis=1).reshape(nw * 3 * npt)
    table = _fuse_table(pe_layer.reshape(1024, EMBED_DIM), point_emb_w,
                        box_emb_w)

    mesh = plsc.VectorSubcoreMesh(core_axis_name="c", subcore_axis_name="s")
    out = pl.kernel(
        _sparse_body,
        out_type=jax.ShapeDtypeStruct((B * _NSLOT, EMBED_DIM), jnp.float32),
        mesh=mesh,
        scratch_types=[
            pltpu.VMEM((3 * npt,), jnp.float32),
            pltpu.VMEM((rows_per_w,), jnp.int32),
            pltpu.VMEM((rows_per_w, EMBED_DIM), jnp.float32),
            pltpu.SemaphoreType.DMA,
        ],
    )(packed, table)
    return out.reshape(B, _NSLOT, EMBED_DIM)


def kernel(point_coords, point_labels, boxes, masks, point_emb_w, box_emb_w,
           conv1_w, conv1_b, conv2_w, conv2_b, conv3_w, conv3_b, pe_layer):
    sparse_embed = _sparse_call(point_coords, point_labels, point_emb_w,
                                box_emb_w, pe_layer)
    dense_embed = _dense_call(masks, conv1_w, conv1_b, conv2_w, conv2_b,
                              conv3_w, conv3_b)
    return (sparse_embed, dense_embed)


# speedup vs baseline: 1.1079x; 1.0216x over previous
import functools

import jax
import jax.numpy as jnp
import numpy as np
from jax import lax
from jax.experimental import pallas as pl
from jax.experimental.pallas import tpu as pltpu
from jax.experimental.pallas import tpu_sc as plsc

EMBED_DIM = 256
IMG_EMB_SIZE = 32


def _dense_body(x_ref, w1_ref, b1_ref, w2_ref, b2_ref,
                w3_ref, b3_ref, out_ref):
    for b in range(x_ref.shape[0]):
        xf = x_ref[b]
        p2k = jnp.maximum(
            jnp.dot(w1_ref[...], xf, preferred_element_type=jnp.float32)
            + b1_ref[...], 0.0)
        h2 = jnp.maximum(
            jnp.dot(w2_ref[...], p2k, preferred_element_type=jnp.float32)
            + b2_ref[...], 0.0)
        out_ref[b] = (jnp.dot(w3_ref[...], h2,
                              preferred_element_type=jnp.float32)
                      + b3_ref[...])


def _dense_call(masks, conv1_w, conv1_b, conv2_w, conv2_b, conv3_w, conv3_b,
                interpret=False):
    B = masks.shape[0]
    xf = masks.reshape(B, 32, 4, 32, 4).transpose(0, 2, 4, 1, 3)
    xf = xf.reshape(B, 16, 1024)

    w1c = conv1_w[:, 0]
    w1big = jnp.stack([
        jnp.pad(w1c, ((0, 0), (2 * di, 2 - 2 * di), (2 * dj, 2 - 2 * dj)))
        for di in range(2) for dj in range(2)
    ], axis=0).reshape(256, 16)
    b1col = jnp.tile(conv1_b, (4,)).reshape(256, 1)
    w2m = conv2_w.transpose(0, 2, 3, 1).reshape(256, 256)
    b2col = conv2_b.reshape(256, 1)
    w3m = conv3_w[:, :, 0, 0]
    b3col = conv3_b.reshape(256, 1)

    const = lambda *_: (0, 0)
    bpb = 4
    out = pl.pallas_call(
        _dense_body,
        grid=(B // bpb,),
        in_specs=[
            pl.BlockSpec((bpb, 16, 1024), lambda i: (i, 0, 0)),
            pl.BlockSpec((256, 16), const),
            pl.BlockSpec((256, 1), const),
            pl.BlockSpec((256, 256), const),
            pl.BlockSpec((256, 1), const),
            pl.BlockSpec((256, 256), const),
            pl.BlockSpec((256, 1), const),
        ],
        out_specs=pl.BlockSpec((bpb, 256, 1024), lambda i: (i, 0, 0)),
        out_shape=jax.ShapeDtypeStruct((B, 256, 1024), jnp.float32),
        interpret=interpret,
    )(xf, w1big, b1col, w2m, b2col, w3m, b3col)
    return out.reshape(B, 256, 32, 32)


_NPOINT = 32
_NBOX = 8
_NSLOT = _NPOINT + _NBOX


def _fuse_body(pe_ref, rows_ref, out_ref):
    mul = jnp.where(pl.program_id(0) == 2, 0.0, 1.0)
    out_ref[0] = pe_ref[...] * mul + rows_ref[0]


def _fuse_table(pe_flat, point_emb_w, box_emb_w, interpret=False):
    rows = jnp.concatenate([point_emb_w, box_emb_w], axis=0)
    rows = rows.reshape(3, 1, EMBED_DIM)
    fused = pl.pallas_call(
        _fuse_body,
        grid=(3,),
        in_specs=[
            pl.BlockSpec((1024, EMBED_DIM), lambda r: (0, 0)),
            pl.BlockSpec((1, 1, EMBED_DIM), lambda r: (r, 0, 0)),
        ],
        out_specs=pl.BlockSpec((1, 1024, EMBED_DIM), lambda r: (r, 0, 0)),
        out_shape=jax.ShapeDtypeStruct((3, 1024, EMBED_DIM), jnp.float32),
        interpret=interpret,
    )(pe_flat, rows)
    return fused.reshape(3 * 1024, EMBED_DIM)


def _sparse_body(pk_hbm, table_hbm, out_hbm, pk_v, idx_v, buf_v, sem):
    nc = 2
    wid = lax.axis_index("s") * nc + lax.axis_index("c")
    pltpu.sync_copy(pk_hbm.at[pl.ds(wid * 192, 192)], pk_v)
    scale = jnp.float32(IMG_EMB_SIZE / 512.0)
    smax = IMG_EMB_SIZE - 1
    box_idx = jnp.full((16,), 2048, jnp.int32)

    def point_chunk(k):
        sl = pl.ds(k * 16, 16)
        xv = pk_v[sl]
        yv = pk_v[pl.ds(64 + k * 16, 16)]
        lv = pk_v[pl.ds(128 + k * 16, 16)].astype(jnp.int32)
        xi = jnp.clip((xv * scale).astype(jnp.int32), 0, smax)
        yi = jnp.clip((yv * scale).astype(jnp.int32), 0, smax)
        return lv * 1024 + yi * IMG_EMB_SIZE + xi

    idx_v[pl.ds(0, 16)] = point_chunk(0)
    idx_v[pl.ds(16, 16)] = point_chunk(1)
    idx_v[pl.ds(32, 16)] = box_idx
    idx_v[pl.ds(40, 16)] = point_chunk(2)
    idx_v[pl.ds(64, 16)] = box_idx
    idx_v[pl.ds(56, 16)] = point_chunk(3)
    copies = [
        pltpu.async_copy(table_hbm.at[idx_v.at[pl.ds(k * 16, 16)]],
                         buf_v.at[pl.ds(k * 16, 16)], sem)
        for k in range(5)
    ]
    for c in copies:
        c.wait()
    pltpu.sync_copy(buf_v, out_hbm.at[pl.ds(wid * 80, 80)])


def _sparse_call(point_coords, point_labels, point_emb_w, box_emb_w, pe_layer):
    B, Np = point_labels.shape
    nw = 32
    rows_per_w = B * _NSLOT // nw
    npt = B * Np // nw
    xs = point_coords[..., 0].reshape(nw, 1, npt)
    ys = point_coords[..., 1].reshape(nw, 1, npt)
    lab = point_labels.astype(jnp.float32).reshape(nw, 1, npt)
    packed = jnp.concatenate([xs, ys, lab], axis=1).reshape(nw * 3 * npt)
    table = _fuse_table(pe_layer.reshape(1024, EMBED_DIM), point_emb_w,
                        box_emb_w)

    mesh = plsc.VectorSubcoreMesh(core_axis_name="c", subcore_axis_name="s")
    out = pl.kernel(
        _sparse_body,
        out_type=jax.ShapeDtypeStruct((B * _NSLOT, EMBED_DIM), jnp.float32),
        mesh=mesh,
        scratch_types=[
            pltpu.VMEM((3 * npt,), jnp.float32),
            pltpu.VMEM((rows_per_w,), jnp.int32),
            pltpu.VMEM((rows_per_w, EMBED_DIM), jnp.float32),
            pltpu.SemaphoreType.DMA,
        ],
    )(packed, table)
    return out.reshape(B, _NSLOT, EMBED_DIM)


def kernel(point_coords, point_labels, boxes, masks, point_emb_w, box_emb_w,
           conv1_w, conv1_b, conv2_w, conv2_b, conv3_w, conv3_b, pe_layer):
    sparse_embed = _sparse_call(point_coords, point_labels, point_emb_w,
                                box_emb_w, pe_layer)
    dense_embed = _dense_call(masks, conv1_w, conv1_b, conv2_w, conv2_b,
                              conv3_w, conv3_b)
    return (sparse_embed, dense_embed)


# speedup vs baseline: 1.1161x; 1.0074x over previous
import functools

import jax
import jax.numpy as jnp
import numpy as np
from jax import lax
from jax.experimental import pallas as pl
from jax.experimental.pallas import tpu as pltpu
from jax.experimental.pallas import tpu_sc as plsc

EMBED_DIM = 256
IMG_EMB_SIZE = 32


def _dense_body(x_ref, w1_ref, b1_ref, w2_ref, b2_ref,
                w3_ref, b3_ref, out_ref):
    for b in range(x_ref.shape[0]):
        xf = x_ref[b]
        p2k = jnp.maximum(
            jnp.dot(w1_ref[...], xf, preferred_element_type=jnp.float32)
            + b1_ref[...], 0.0)
        h2 = jnp.maximum(
            jnp.dot(w2_ref[...], p2k, preferred_element_type=jnp.float32)
            + b2_ref[...], 0.0)
        out_ref[b] = (jnp.dot(w3_ref[...], h2,
                              preferred_element_type=jnp.float32)
                      + b3_ref[...])


def _dense_call(masks, conv1_w, conv1_b, conv2_w, conv2_b, conv3_w, conv3_b,
                interpret=False):
    B = masks.shape[0]
    xf = masks.reshape(B, 32, 4, 32, 4).transpose(0, 2, 4, 1, 3)
    xf = xf.reshape(B, 16, 1024)

    w1c = conv1_w[:, 0]
    w1big = jnp.stack([
        jnp.pad(w1c, ((0, 0), (2 * di, 2 - 2 * di), (2 * dj, 2 - 2 * dj)))
        for di in range(2) for dj in range(2)
    ], axis=0).reshape(256, 16)
    b1col = jnp.tile(conv1_b, (4,)).reshape(256, 1)
    w2m = conv2_w.transpose(0, 2, 3, 1).reshape(256, 256)
    b2col = conv2_b.reshape(256, 1)
    w3m = conv3_w[:, :, 0, 0]
    b3col = conv3_b.reshape(256, 1)

    const = lambda *_: (0, 0)
    bpb = 8
    out = pl.pallas_call(
        _dense_body,
        grid=(B // bpb,),
        in_specs=[
            pl.BlockSpec((bpb, 16, 1024), lambda i: (i, 0, 0)),
            pl.BlockSpec((256, 16), const),
            pl.BlockSpec((256, 1), const),
            pl.BlockSpec((256, 256), const),
            pl.BlockSpec((256, 1), const),
            pl.BlockSpec((256, 256), const),
            pl.BlockSpec((256, 1), const),
        ],
        out_specs=pl.BlockSpec((bpb, 256, 1024), lambda i: (i, 0, 0)),
        out_shape=jax.ShapeDtypeStruct((B, 256, 1024), jnp.float32),
        interpret=interpret,
    )(xf, w1big, b1col, w2m, b2col, w3m, b3col)
    return out.reshape(B, 256, 32, 32)


_NPOINT = 32
_NBOX = 8
_NSLOT = _NPOINT + _NBOX


def _fuse_body(pe_ref, rows_ref, out_ref):
    mul = jnp.where(pl.program_id(0) == 2, 0.0, 1.0)
    out_ref[0] = pe_ref[...] * mul + rows_ref[0]


def _fuse_table(pe_flat, point_emb_w, box_emb_w, interpret=False):
    rows = jnp.concatenate([point_emb_w, box_emb_w], axis=0)
    rows = rows.reshape(3, 1, EMBED_DIM)
    fused = pl.pallas_call(
        _fuse_body,
        grid=(3,),
        in_specs=[
            pl.BlockSpec((1024, EMBED_DIM), lambda r: (0, 0)),
            pl.BlockSpec((1, 1, EMBED_DIM), lambda r: (r, 0, 0)),
        ],
        out_specs=pl.BlockSpec((1, 1024, EMBED_DIM), lambda r: (r, 0, 0)),
        out_shape=jax.ShapeDtypeStruct((3, 1024, EMBED_DIM), jnp.float32),
        interpret=interpret,
    )(pe_flat, rows)
    return fused.reshape(3 * 1024, EMBED_DIM)


def _sparse_body(pk_hbm, table_hbm, out_hbm, pk_v, idx_v, buf_v, sem):
    nc = 2
    wid = lax.axis_index("s") * nc + lax.axis_index("c")
    pltpu.sync_copy(pk_hbm.at[pl.ds(wid * 192, 192)], pk_v)
    scale = jnp.float32(IMG_EMB_SIZE / 512.0)
    smax = IMG_EMB_SIZE - 1
    box_idx = jnp.full((16,), 2048, jnp.int32)

    def point_chunk(k):
        sl = pl.ds(k * 16, 16)
        xv = pk_v[sl]
        yv = pk_v[pl.ds(64 + k * 16, 16)]
        lv = pk_v[pl.ds(128 + k * 16, 16)].astype(jnp.int32)
        xi = jnp.clip((xv * scale).astype(jnp.int32), 0, smax)
        yi = jnp.clip((yv * scale).astype(jnp.int32), 0, smax)
        return lv * 1024 + yi * IMG_EMB_SIZE + xi

    idx_v[pl.ds(0, 16)] = point_chunk(0)
    idx_v[pl.ds(16, 16)] = point_chunk(1)
    idx_v[pl.ds(32, 16)] = box_idx
    idx_v[pl.ds(40, 16)] = point_chunk(2)
    idx_v[pl.ds(64, 16)] = box_idx
    idx_v[pl.ds(56, 16)] = point_chunk(3)
    copies = [
        pltpu.async_copy(table_hbm.at[idx_v.at[pl.ds(k * 16, 16)]],
                         buf_v.at[pl.ds(k * 16, 16)], sem)
        for k in range(5)
    ]
    for c in copies:
        c.wait()
    pltpu.sync_copy(buf_v, out_hbm.at[pl.ds(wid * 80, 80)])


def _sparse_call(point_coords, point_labels, point_emb_w, box_emb_w, pe_layer):
    B, Np = point_labels.shape
    nw = 32
    rows_per_w = B * _NSLOT // nw
    npt = B * Np // nw
    xs = point_coords[..., 0].reshape(nw, 1, npt)
    ys = point_coords[..., 1].reshape(nw, 1, npt)
    lab = point_labels.astype(jnp.float32).reshape(nw, 1, npt)
    packed = jnp.concatenate([xs, ys, lab], axis=1).reshape(nw * 3 * npt)
    table = _fuse_table(pe_layer.reshape(1024, EMBED_DIM), point_emb_w,
                        box_emb_w)

    mesh = plsc.VectorSubcoreMesh(core_axis_name="c", subcore_axis_name="s")
    out = pl.kernel(
        _sparse_body,
        out_type=jax.ShapeDtypeStruct((B * _NSLOT, EMBED_DIM), jnp.float32),
        mesh=mesh,
        scratch_types=[
            pltpu.VMEM((3 * npt,), jnp.float32),
            pltpu.VMEM((rows_per_w,), jnp.int32),
            pltpu.VMEM((rows_per_w, EMBED_DIM), jnp.float32),
            pltpu.SemaphoreType.DMA,
        ],
    )(packed, table)
    return out.reshape(B, _NSLOT, EMBED_DIM)


def kernel(point_coords, point_labels, boxes, masks, point_emb_w, box_emb_w,
           conv1_w, conv1_b, conv2_w, conv2_b, conv3_w, conv3_b, pe_layer):
    sparse_embed = _sparse_call(point_coords, point_labels, point_emb_w,
                                box_emb_w, pe_layer)
    dense_embed = _dense_call(masks, conv1_w, conv1_b, conv2_w, conv2_b,
                              conv3_w, conv3_b)
    return (sparse_embed, dense_embed)


# speedup vs baseline: 1.1905x; 1.0667x over previous
import functools

import jax
import jax.numpy as jnp
import numpy as np
from jax import lax
from jax.experimental import pallas as pl
from jax.experimental.pallas import tpu as pltpu
from jax.experimental.pallas import tpu_sc as plsc

EMBED_DIM = 256
IMG_EMB_SIZE = 32


def _dense_body(x_ref, w1_ref, b1_ref, w2_ref, b2_ref,
                w3_ref, b3_ref, out_ref):
    for b in range(x_ref.shape[0]):
        xf = x_ref[b]
        p2k = jnp.maximum(
            jnp.dot(w1_ref[...], xf, preferred_element_type=jnp.float32)
            + b1_ref[...], 0.0)
        h2 = jnp.maximum(
            jnp.dot(w2_ref[...], p2k, preferred_element_type=jnp.float32)
            + b2_ref[...], 0.0)
        h3 = (jnp.dot(w3_ref[...], h2, preferred_element_type=jnp.float32)
              + b3_ref[...])
        out_ref[b] = h3.astype(jnp.bfloat16)


def _dense_call(masks, conv1_w, conv1_b, conv2_w, conv2_b, conv3_w, conv3_b,
                interpret=False):
    B = masks.shape[0]
    xf = masks.reshape(B, 32, 4, 32, 4).transpose(0, 2, 4, 1, 3)
    xf = xf.reshape(B, 16, 1024)

    w1c = conv1_w[:, 0]
    w1big = jnp.stack([
        jnp.pad(w1c, ((0, 0), (2 * di, 2 - 2 * di), (2 * dj, 2 - 2 * dj)))
        for di in range(2) for dj in range(2)
    ], axis=0).reshape(256, 16)
    b1col = jnp.tile(conv1_b, (4,)).reshape(256, 1)
    w2m = conv2_w.transpose(0, 2, 3, 1).reshape(256, 256)
    b2col = conv2_b.reshape(256, 1)
    w3m = conv3_w[:, :, 0, 0]
    b3col = conv3_b.reshape(256, 1)

    const = lambda *_: (0, 0)
    bpb = 8
    out = pl.pallas_call(
        _dense_body,
        grid=(B // bpb,),
        in_specs=[
            pl.BlockSpec((bpb, 16, 1024), lambda i: (i, 0, 0)),
            pl.BlockSpec((256, 16), const),
            pl.BlockSpec((256, 1), const),
            pl.BlockSpec((256, 256), const),
            pl.BlockSpec((256, 1), const),
            pl.BlockSpec((256, 256), const),
            pl.BlockSpec((256, 1), const),
        ],
        out_specs=pl.BlockSpec((bpb, 256, 1024), lambda i: (i, 0, 0)),
        out_shape=jax.ShapeDtypeStruct((B, 256, 1024), jnp.bfloat16),
        interpret=interpret,
    )(xf, w1big, b1col, w2m, b2col, w3m, b3col)
    return out.reshape(B, 256, 32, 32).astype(jnp.float32)


_NPOINT = 32
_NBOX = 8
_NSLOT = _NPOINT + _NBOX


def _fuse_body(pe_ref, rows_ref, out_ref):
    mul = jnp.where(pl.program_id(0) == 2, 0.0, 1.0)
    out_ref[0] = pe_ref[...] * mul + rows_ref[0]


def _fuse_table(pe_flat, point_emb_w, box_emb_w, interpret=False):
    rows = jnp.concatenate([point_emb_w, box_emb_w], axis=0)
    rows = rows.reshape(3, 1, EMBED_DIM)
    fused = pl.pallas_call(
        _fuse_body,
        grid=(3,),
        in_specs=[
            pl.BlockSpec((1024, EMBED_DIM), lambda r: (0, 0)),
            pl.BlockSpec((1, 1, EMBED_DIM), lambda r: (r, 0, 0)),
        ],
        out_specs=pl.BlockSpec((1, 1024, EMBED_DIM), lambda r: (r, 0, 0)),
        out_shape=jax.ShapeDtypeStruct((3, 1024, EMBED_DIM), jnp.float32),
        interpret=interpret,
    )(pe_flat, rows)
    return fused.reshape(3 * 1024, EMBED_DIM)


def _sparse_body(pk_hbm, table_hbm, out_hbm, pk_v, idx_v, buf_v, sem):
    nc = 2
    wid = lax.axis_index("s") * nc + lax.axis_index("c")
    pltpu.sync_copy(pk_hbm.at[pl.ds(wid * 192, 192)], pk_v)
    scale = jnp.float32(IMG_EMB_SIZE / 512.0)
    smax = IMG_EMB_SIZE - 1
    box_idx = jnp.full((16,), 2048, jnp.int32)

    def point_chunk(k):
        sl = pl.ds(k * 16, 16)
        xv = pk_v[sl]
        yv = pk_v[pl.ds(64 + k * 16, 16)]
        lv = pk_v[pl.ds(128 + k * 16, 16)].astype(jnp.int32)
        xi = jnp.clip((xv * scale).astype(jnp.int32), 0, smax)
        yi = jnp.clip((yv * scale).astype(jnp.int32), 0, smax)
        return lv * 1024 + yi * IMG_EMB_SIZE + xi

    idx_v[pl.ds(0, 16)] = point_chunk(0)
    idx_v[pl.ds(16, 16)] = point_chunk(1)
    idx_v[pl.ds(32, 16)] = box_idx
    idx_v[pl.ds(40, 16)] = point_chunk(2)
    idx_v[pl.ds(64, 16)] = box_idx
    idx_v[pl.ds(56, 16)] = point_chunk(3)
    copies = [
        pltpu.async_copy(table_hbm.at[idx_v.at[pl.ds(k * 16, 16)]],
                         buf_v.at[pl.ds(k * 16, 16)], sem)
        for k in range(5)
    ]
    for c in copies:
        c.wait()
    pltpu.sync_copy(buf_v, out_hbm.at[pl.ds(wid * 80, 80)])


def _sparse_call(point_coords, point_labels, point_emb_w, box_emb_w, pe_layer):
    B, Np = point_labels.shape
    nw = 32
    rows_per_w = B * _NSLOT // nw
    npt = B * Np // nw
    xs = point_coords[..., 0].reshape(nw, 1, npt)
    ys = point_coords[..., 1].reshape(nw, 1, npt)
    lab = point_labels.astype(jnp.float32).reshape(nw, 1, npt)
    packed = jnp.concatenate([xs, ys, lab], axis=1).reshape(nw * 3 * npt)
    table = _fuse_table(pe_layer.reshape(1024, EMBED_DIM), point_emb_w,
                        box_emb_w)

    mesh = plsc.VectorSubcoreMesh(core_axis_name="c", subcore_axis_name="s")
    out = pl.kernel(
        _sparse_body,
        out_type=jax.ShapeDtypeStruct((B * _NSLOT, EMBED_DIM), jnp.float32),
        mesh=mesh,
        scratch_types=[
            pltpu.VMEM((3 * npt,), jnp.float32),
            pltpu.VMEM((rows_per_w,), jnp.int32),
            pltpu.VMEM((rows_per_w, EMBED_DIM), jnp.float32),
            pltpu.SemaphoreType.DMA,
        ],
    )(packed, table)
    return out.reshape(B, _NSLOT, EMBED_DIM)


def kernel(point_coords, point_labels, boxes, masks, point_emb_w, box_emb_w,
           conv1_w, conv1_b, conv2_w, conv2_b, conv3_w, conv3_b, pe_layer):
    sparse_embed = _sparse_call(point_coords, point_labels, point_emb_w,
                                box_emb_w, pe_layer)
    dense_embed = _dense_call(masks, conv1_w, conv1_b, conv2_w, conv2_b,
                              conv3_w, conv3_b)
    return (sparse_embed, dense_embed)
